# baseline (speedup 1.0000x reference)
import jax
import jax.numpy as jnp
from jax.experimental import pallas as pl
from jax.experimental.pallas import tpu as pltpu


def _round_up(x, m):
    return (x + m - 1) // m * m


def kernel(x_nchw, wconv_pt, bconv, whead_pt, bhead):
    B, C, H, W = x_nchw.shape
    F = wconv_pt.shape[0]
    n_class = whead_pt.shape[0]
    R = H * W
    KC = 3 * C
    KL = _round_up(KC + 1, 16)
    K = 3 * KL
    F_pad = _round_up(F, 128)
    C_pad = _round_up(n_class, 128)
    G = 4
    assert B % G == 0

    x_nhwc = jnp.transpose(x_nchw, (0, 2, 3, 1))
    xpw = jnp.pad(x_nhwc, ((0, 0), (0, 0), (1, 1), (0, 0)))
    taps = jnp.stack([xpw[:, :, kx:kx + W, :] for kx in range(3)], axis=3)
    xrow = taps.reshape(B, R, KC)
    xrow = jnp.concatenate(
        [xrow, jnp.zeros((B, R, KL - KC - 1), xrow.dtype),
         jnp.ones((B, R, 1), xrow.dtype)], axis=2)
    xrow = jnp.pad(xrow, ((0, 0), (W, W), (0, 0)))
    xrow = xrow.astype(jnp.bfloat16)

    wk = jnp.transpose(wconv_pt, (2, 3, 1, 0)).reshape(3, KC, F)
    wk = jnp.pad(wk, ((0, 0), (0, KL - KC), (0, 0)))
    wk = wk.at[1, KL - 1, :].set(bconv)
    wk = jnp.pad(wk.reshape(K, F), ((0, 0), (0, F_pad - F))).astype(jnp.bfloat16)

    whead = (jnp.transpose(whead_pt, (1, 0)) / float(R))
    whead = jnp.pad(whead, ((0, F_pad - F),
                            (0, C_pad - n_class))).astype(jnp.float32)
    bhead_p = jnp.pad(bhead.reshape(1, n_class),
                      ((0, 0), (0, C_pad - n_class))).astype(jnp.float32)

    def _body(xr_ref, w_ref, wh_ref, bh_ref, out_ref):
        for g in range(G):
            blk = xr_ref[g]
            patches = jnp.concatenate(
                [blk[0:R], blk[W:W + R], blk[2 * W:2 * W + R]], axis=1)
            conv = jnp.dot(patches, w_ref[...],
                           preferred_element_type=jnp.float32)
            conv = jnp.maximum(conv, 0.0)
            pooled = jnp.sum(conv, axis=0, keepdims=True)
            logits = jnp.dot(pooled, wh_ref[...],
                             preferred_element_type=jnp.float32) + bh_ref[...]
            out_ref[g] = logits

    flops = 2 * B * R * K * F_pad + 2 * B * F_pad * C_pad
    bytes_accessed = (xrow.size * 2 + wk.size * 2
                      + (whead.size + bhead_p.size) * 4 + B * C_pad * 4)

    out = pl.pallas_call(
        _body,
        out_shape=jax.ShapeDtypeStruct((B, 1, C_pad), jnp.float32),
        grid=(B // G,),
        in_specs=[
            pl.BlockSpec((G, R + 2 * W, KL), lambda i: (i, 0, 0)),
            pl.BlockSpec((K, F_pad), lambda i: (0, 0)),
            pl.BlockSpec((F_pad, C_pad), lambda i: (0, 0)),
            pl.BlockSpec((1, C_pad), lambda i: (0, 0)),
        ],
        out_specs=pl.BlockSpec((G, 1, C_pad), lambda i: (i, 0, 0)),
        compiler_params=pltpu.CompilerParams(
            dimension_semantics=("parallel",),
            vmem_limit_bytes=48 * 1024 * 1024,
        ),
        cost_estimate=pl.CostEstimate(
            flops=flops, transcendentals=0, bytes_accessed=bytes_accessed),
    )(xrow, wk, whead, bhead_p)

    return out[:, 0, :n_class]


# speedup vs baseline: 1.0256x; 1.0256x over previous
import jax
import jax.numpy as jnp
from jax.experimental import pallas as pl
from jax.experimental.pallas import tpu as pltpu


def _round_up(x, m):
    return (x + m - 1) // m * m


def kernel(x_nchw, wconv_pt, bconv, whead_pt, bhead):
    B, C, H, W = x_nchw.shape
    F = wconv_pt.shape[0]
    n_class = whead_pt.shape[0]
    R = H * W
    KC = 3 * C
    KL = _round_up(KC + 1, 16)
    K = 3 * KL
    F_pad = _round_up(F, 128)
    C_pad = _round_up(n_class, 128)
    G = 1
    assert B % G == 0

    x_nhwc = jnp.transpose(x_nchw, (0, 2, 3, 1))
    xpw = jnp.pad(x_nhwc, ((0, 0), (0, 0), (1, 1), (0, 0)))
    taps = jnp.stack([xpw[:, :, kx:kx + W, :] for kx in range(3)], axis=3)
    xrow = taps.reshape(B, R, KC)
    xrow = jnp.concatenate(
        [xrow, jnp.zeros((B, R, KL - KC - 1), xrow.dtype),
         jnp.ones((B, R, 1), xrow.dtype)], axis=2)
    xrow = jnp.pad(xrow, ((0, 0), (W, W), (0, 0)))
    xrow = xrow.astype(jnp.bfloat16)

    wk = jnp.transpose(wconv_pt, (2, 3, 1, 0)).reshape(3, KC, F)
    wk = jnp.pad(wk, ((0, 0), (0, KL - KC), (0, 0)))
    wk = wk.at[1, KL - 1, :].set(bconv)
    wk = jnp.pad(wk.reshape(K, F), ((0, 0), (0, F_pad - F))).astype(jnp.bfloat16)

    whead = (jnp.transpose(whead_pt, (1, 0)) / float(R))
    whead = jnp.pad(whead, ((0, F_pad - F),
                            (0, C_pad - n_class))).astype(jnp.float32)
    bhead_p = jnp.pad(bhead.reshape(1, n_class),
                      ((0, 0), (0, C_pad - n_class))).astype(jnp.float32)

    def _body(xr_ref, w_ref, wh_ref, bh_ref, out_ref):
        for g in range(G):
            blk = xr_ref[g]
            patches = jnp.concatenate(
                [blk[0:R], blk[W:W + R], blk[2 * W:2 * W + R]], axis=1)
            conv = jnp.dot(patches, w_ref[...],
                           preferred_element_type=jnp.float32)
            conv = jnp.maximum(conv, 0.0)
            pooled = jnp.sum(conv, axis=0, keepdims=True)
            logits = jnp.dot(pooled, wh_ref[...],
                             preferred_element_type=jnp.float32) + bh_ref[...]
            out_ref[g] = logits

    flops = 2 * B * R * K * F_pad + 2 * B * F_pad * C_pad
    bytes_accessed = (xrow.size * 2 + wk.size * 2
                      + (whead.size + bhead_p.size) * 4 + B * C_pad * 4)

    out = pl.pallas_call(
        _body,
        out_shape=jax.ShapeDtypeStruct((B, 1, C_pad), jnp.float32),
        grid=(B // G,),
        in_specs=[
            pl.BlockSpec((G, R + 2 * W, KL), lambda i: (i, 0, 0)),
            pl.BlockSpec((K, F_pad), lambda i: (0, 0)),
            pl.BlockSpec((F_pad, C_pad), lambda i: (0, 0)),
            pl.BlockSpec((1, C_pad), lambda i: (0, 0)),
        ],
        out_specs=pl.BlockSpec((G, 1, C_pad), lambda i: (i, 0, 0)),
        compiler_params=pltpu.CompilerParams(
            dimension_semantics=("parallel",),
            vmem_limit_bytes=48 * 1024 * 1024,
        ),
        cost_estimate=pl.CostEstimate(
            flops=flops, transcendentals=0, bytes_accessed=bytes_accessed),
    )(xrow, wk, whead, bhead_p)

    return out[:, 0, :n_class]


# speedup vs baseline: 1.9060x; 1.8585x over previous
import jax
import jax.numpy as jnp
from jax.experimental import pallas as pl
from jax.experimental.pallas import tpu as pltpu


def _round_up(x, m):
    return (x + m - 1) // m * m


def kernel(x_nchw, wconv_pt, bconv, whead_pt, bhead):
    B, C, H, W = x_nchw.shape
    F = wconv_pt.shape[0]
    n_class = whead_pt.shape[0]
    R = H * W
    CP = 8
    K = 9 * CP
    F_pad = _round_up(F, 128)
    C_pad = _round_up(n_class, 128)
    PAD = 128

    xflat = x_nchw.reshape(B, C, R).astype(jnp.bfloat16)
    xflat = jnp.pad(xflat, ((0, 0), (0, CP - C), (PAD, PAD)))

    w_of_r = jax.lax.broadcasted_iota(jnp.int32, (CP, R), 1) % W
    mask0 = (w_of_r != 0).astype(jnp.bfloat16)
    mask2 = (w_of_r != W - 1).astype(jnp.bfloat16)
    row_of = jax.lax.broadcasted_iota(jnp.int32, (CP, R), 0)
    carrier = (row_of == CP - 1).astype(jnp.bfloat16)

    wk = jnp.transpose(wconv_pt, (2, 3, 1, 0)).reshape(9, C, F)
    wk = jnp.pad(wk, ((0, 0), (0, CP - C), (0, 0)))
    wk = wk.at[4, CP - 1, :].set(bconv)
    wk = jnp.pad(wk.reshape(K, F), ((0, 0), (0, F_pad - F)))
    wk = jnp.transpose(wk, (1, 0)).astype(jnp.bfloat16)

    whead = (jnp.transpose(whead_pt, (1, 0)) / float(R))
    whead = jnp.pad(whead, ((0, F_pad - F),
                            (0, C_pad - n_class))).astype(jnp.float32)
    bhead_p = jnp.pad(bhead.reshape(1, n_class),
                      ((0, 0), (0, C_pad - n_class))).astype(jnp.float32)

    def _body(x_ref, w_ref, m0_ref, m2_ref, car_ref, wh_ref, bh_ref,
              out_ref, pt_ref):
        xb = x_ref[0]
        for t in range(9):
            ky, kx = divmod(t, 3)
            d = PAD + (ky - 1) * W + (kx - 1)
            slab = xb[:, d:d + R]
            if kx == 0:
                slab = slab * m0_ref[...]
            elif kx == 2:
                slab = slab * m2_ref[...]
            else:
                if ky == 1:
                    slab = slab + car_ref[...]
            pt_ref[CP * t:CP * (t + 1), :] = slab
        conv = jnp.dot(w_ref[...], pt_ref[...],
                       preferred_element_type=jnp.float32)
        conv = jnp.maximum(conv, 0.0)
        pooled = jnp.sum(conv, axis=1, keepdims=True)
        logits = jax.lax.dot_general(
            pooled, wh_ref[...], (((0,), (0,)), ((), ())),
            preferred_element_type=jnp.float32) + bh_ref[...]
        out_ref[0] = logits

    flops = 2 * B * R * K * F_pad + 2 * B * F_pad * C_pad
    bytes_accessed = (xflat.size * 2 + wk.size * 2
                      + (whead.size + bhead_p.size) * 4 + B * C_pad * 4)

    out = pl.pallas_call(
        _body,
        out_shape=jax.ShapeDtypeStruct((B, 1, C_pad), jnp.float32),
        grid=(B,),
        in_specs=[
            pl.BlockSpec((1, CP, R + 2 * PAD), lambda b: (b, 0, 0)),
            pl.BlockSpec((F_pad, K), lambda b: (0, 0)),
            pl.BlockSpec((CP, R), lambda b: (0, 0)),
            pl.BlockSpec((CP, R), lambda b: (0, 0)),
            pl.BlockSpec((CP, R), lambda b: (0, 0)),
            pl.BlockSpec((F_pad, C_pad), lambda b: (0, 0)),
            pl.BlockSpec((1, C_pad), lambda b: (0, 0)),
        ],
        out_specs=pl.BlockSpec((1, 1, C_pad), lambda b: (b, 0, 0)),
        scratch_shapes=[pltpu.VMEM((K, R), jnp.bfloat16)],
        compiler_params=pltpu.CompilerParams(
            dimension_semantics=("parallel",),
            vmem_limit_bytes=48 * 1024 * 1024,
        ),
        cost_estimate=pl.CostEstimate(
            flops=flops, transcendentals=0, bytes_accessed=bytes_accessed),
    )(xflat, wk, mask0, mask2, carrier, whead, bhead_p)

    return out[:, 0, :n_class]


# speedup vs baseline: 2.6087x; 1.3686x over previous
import jax
import jax.numpy as jnp
from jax.experimental import pallas as pl
from jax.experimental.pallas import tpu as pltpu


def _round_up(x, m):
    return (x + m - 1) // m * m


def kernel(x_nchw, wconv_pt, bconv, whead_pt, bhead):
    B, C, H, W = x_nchw.shape
    F = wconv_pt.shape[0]
    n_class = whead_pt.shape[0]
    R = H * W
    CP = 8
    K = 10 * CP
    F_pad = _round_up(F, 128)
    C_pad = _round_up(n_class, 128)

    xflat = jnp.reshape(x_nchw.astype(jnp.bfloat16), (B, C, R))

    h_of_r = jax.lax.broadcasted_iota(jnp.int32, (9, CP, R), 2) // W
    w_of_r = jax.lax.broadcasted_iota(jnp.int32, (9, CP, R), 2) % W
    ky_t = jax.lax.broadcasted_iota(jnp.int32, (9, CP, R), 0) // 3
    kx_t = jax.lax.broadcasted_iota(jnp.int32, (9, CP, R), 0) % 3
    hh = h_of_r + ky_t - 1
    ww = w_of_r + kx_t - 1
    masks = ((hh >= 0) & (hh < H) & (ww >= 0) & (ww < W)).astype(jnp.bfloat16)

    row_of = jax.lax.broadcasted_iota(jnp.int32, (CP, R), 0)
    carrier = (row_of == CP - 1).astype(jnp.bfloat16)

    wk = jnp.transpose(wconv_pt, (2, 3, 1, 0)).reshape(9, C, F)
    wk = jnp.pad(wk, ((0, 1), (0, CP - C), (0, 0)))
    wk = wk.at[9, CP - 1, :].set(bconv)
    wk = jnp.pad(wk.reshape(K, F), ((0, 0), (0, F_pad - F)))
    wk = jnp.transpose(wk, (1, 0)).astype(jnp.bfloat16)

    whead = (jnp.transpose(whead_pt, (1, 0)) / float(R))
    whead = jnp.pad(whead, ((0, F_pad - F),
                            (0, C_pad - n_class))).astype(jnp.float32)
    bhead_p = jnp.pad(bhead.reshape(1, n_class),
                      ((0, 0), (0, C_pad - n_class))).astype(jnp.float32)

    def _body(x_ref, w_ref, m_ref, car_ref, wh_ref, bh_ref, out_ref, pt_ref):
        xb = x_ref[0]
        pt_ref[...] = jnp.zeros_like(pt_ref)
        for t in range(9):
            ky, kx = divmod(t, 3)
            s = ((ky - 1) * W + (kx - 1)) % R
            if s == 0:
                slab = xb * m_ref[t, 0:C]
            else:
                slab = jnp.concatenate([xb[:, s:], xb[:, :s]], axis=1)
                slab = slab * m_ref[t, 0:C]
            pt_ref[CP * t:CP * t + C, :] = slab
        pt_ref[CP * 9:CP * 10, :] = car_ref[...]
        conv = jnp.dot(w_ref[...], pt_ref[...],
                       preferred_element_type=jnp.float32)
        conv = jnp.maximum(conv, 0.0)
        pooled = jnp.sum(conv, axis=1, keepdims=True)
        logits = jax.lax.dot_general(
            pooled, wh_ref[...], (((0,), (0,)), ((), ())),
            preferred_element_type=jnp.float32) + bh_ref[...]
        out_ref[0] = logits

    flops = 2 * B * R * K * F_pad + 2 * B * F_pad * C_pad
    bytes_accessed = (xflat.size * 2 + wk.size * 2
                      + (whead.size + bhead_p.size) * 4 + B * C_pad * 4)

    out = pl.pallas_call(
        _body,
        out_shape=jax.ShapeDtypeStruct((B, 1, C_pad), jnp.float32),
        grid=(B,),
        in_specs=[
            pl.BlockSpec((1, C, R), lambda b: (b, 0, 0)),
            pl.BlockSpec((F_pad, K), lambda b: (0, 0)),
            pl.BlockSpec((9, CP, R), lambda b: (0, 0, 0)),
            pl.BlockSpec((CP, R), lambda b: (0, 0)),
            pl.BlockSpec((F_pad, C_pad), lambda b: (0, 0)),
            pl.BlockSpec((1, C_pad), lambda b: (0, 0)),
        ],
        out_specs=pl.BlockSpec((1, 1, C_pad), lambda b: (b, 0, 0)),
        scratch_shapes=[pltpu.VMEM((K, R), jnp.bfloat16)],
        compiler_params=pltpu.CompilerParams(
            dimension_semantics=("parallel",),
            vmem_limit_bytes=48 * 1024 * 1024,
        ),
        cost_estimate=pl.CostEstimate(
            flops=flops, transcendentals=0, bytes_accessed=bytes_accessed),
    )(xflat, wk, masks, carrier, whead, bhead_p)

    return out[:, 0, :n_class]
